# baseline (speedup 1.0000x reference)
import functools

import jax
import jax.numpy as jnp
from jax import lax
from jax.experimental import pallas as pl
from jax.experimental.pallas import tpu as pltpu, tpu_sc as plsc

ND = 50000
NM = 2048
ICH = 32
OCH = 64
KNN = 32
RADSQ = 1.0
NB = 8

NWORK = 32
MPW = NM // NWORK
CHUNK = 4096
CAP = 128
NL = 16

_GDN = lax.GatherDimensionNumbers(
    offset_dims=(), collapsed_slice_dims=(0,), start_index_map=(0,))


def _lane_bcast(vec, j):
    idx = jnp.broadcast_to(j.astype(jnp.int32), (NL,))
    return lax.gather(vec, idx[:, None], _GDN, slice_sizes=(1,),
                      mode=lax.GatherScatterMode.PROMISE_IN_BOUNDS)


def _minkv(ak, av, bk, bv):
    sel = ak <= bk
    return jnp.where(sel, ak, bk), jnp.where(sel, av, bv)


def _maxkv(ak, av, bk, bv):
    sel = ak <= bk
    return jnp.where(sel, bk, ak), jnp.where(sel, bv, av)


def _rev(x):
    return lax.rev(x, (0,))


def _merge16(ak, av, bk, bv):
    rbk, rbv = _rev(bk), _rev(bv)
    lok, lov = _minkv(ak, av, rbk, rbv)
    hik, hiv = _maxkv(ak, av, rbk, rbv)
    lok, lov = plsc.sort_key_val(lok, lov)
    hik, hiv = plsc.sort_key_val(hik, hiv)
    return lok, lov, hik, hiv


def _merge32_low(a, b):
    a0k, a0v, a1k, a1v = a
    b0k, b0v, b1k, b1v = b
    l0k, l0v = _minkv(a0k, a0v, _rev(b1k), _rev(b1v))
    l1k, l1v = _minkv(a1k, a1v, _rev(b0k), _rev(b0v))
    m0k, m0v = _minkv(l0k, l0v, l1k, l1v)
    m1k, m1v = _maxkv(l0k, l0v, l1k, l1v)
    m0k, m0v = plsc.sort_key_val(m0k, m0v)
    m1k, m1v = plsc.sort_key_val(m1k, m1v)
    return m0k, m0v, m1k, m1v


def _sc_knn(bxs, bys, bzs, n2s, xs, ys, zs, xm, ym, zm, mbx, mby, mbz, mp2, bm, bnd, xd):
    mesh = plsc.VectorSubcoreMesh(core_axis_name="c", subcore_axis_name="s")
    f32 = jnp.float32
    i32 = jnp.int32

    @functools.partial(
        pl.kernel,
        mesh=mesh,
        compiler_params=pltpu.CompilerParams(needs_layout_passes=False,
                                             use_tc_tiling_on_sc=False),
        out_type=(
            jax.ShapeDtypeStruct((KNN * NM, ICH), f32),
            jax.ShapeDtypeStruct((KNN * NM, 4), f32),
            jax.ShapeDtypeStruct((KNN * NM,), i32),
            jax.ShapeDtypeStruct((KNN * NM,), f32),
        ),
        scratch_types=[
            pltpu.VMEM((CHUNK,), f32),
            pltpu.VMEM((CHUNK,), f32),
            pltpu.VMEM((CHUNK,), f32),
            pltpu.VMEM((CHUNK,), f32),
            pltpu.VMEM((MPW + NL,), f32),
            pltpu.VMEM((MPW + NL,), f32),
            pltpu.VMEM((MPW + NL,), f32),
            pltpu.VMEM((MPW + NL,), f32),
            pltpu.VMEM((MPW + NL,), f32),
            pltpu.VMEM((MPW + NL,), f32),
            pltpu.VMEM((MPW + NL,), f32),
            pltpu.VMEM((MPW + NL,), i32),
            pltpu.VMEM((NL,), i32),
            pltpu.VMEM((CAP,), f32),
            pltpu.VMEM((CAP,), i32),
            pltpu.VMEM((KNN * MPW,), f32),
            pltpu.VMEM((KNN * MPW,), i32),
            pltpu.VMEM((KNN * MPW, ICH), f32),
            pltpu.VMEM((KNN * MPW, 4), f32),
            pltpu.VMEM((KNN * MPW,), f32),
            pltpu.VMEM((KNN * MPW,), f32),
            pltpu.VMEM((KNN * MPW,), f32),
            pltpu.SemaphoreType.DMA,
            pltpu.SemaphoreType.DMA,
        ],
    )
    def knn(bxs_h, bys_h, bzs_h, n2s_h, xs_h, ys_h, zs_h,
            xm_h, ym_h, zm_h, mbx_h, mby_h, mbz_h, mp2_h, bm_h, bnd_h, xd_h,
            efx_h, efr_h, ti_h, td_h,
            cx, cy, cz, cn, mx, my, mz, vbx, vby, vbz, vp2, mb,
            bndv, dbuf, ibuf, topd, topi,
            efx_stage, efr_stage, gx, gy, gz, sem1, sem2):
        wid = lax.axis_index("s") * 2 + lax.axis_index("c")
        mbase = wid * MPW
        iota = lax.iota(i32, NL)
        inf = f32(jnp.inf)

        pltpu.sync_copy(xm_h.at[pl.ds(mbase, MPW)], mx.at[pl.ds(0, MPW)])
        pltpu.sync_copy(ym_h.at[pl.ds(mbase, MPW)], my.at[pl.ds(0, MPW)])
        pltpu.sync_copy(zm_h.at[pl.ds(mbase, MPW)], mz.at[pl.ds(0, MPW)])
        pltpu.sync_copy(mbx_h.at[pl.ds(mbase, MPW)], vbx.at[pl.ds(0, MPW)])
        pltpu.sync_copy(mby_h.at[pl.ds(mbase, MPW)], vby.at[pl.ds(0, MPW)])
        pltpu.sync_copy(mbz_h.at[pl.ds(mbase, MPW)], vbz.at[pl.ds(0, MPW)])
        pltpu.sync_copy(mp2_h.at[pl.ds(mbase, MPW)], vp2.at[pl.ds(0, MPW)])
        pltpu.sync_copy(bm_h.at[pl.ds(mbase, MPW)], mb.at[pl.ds(0, MPW)])
        pltpu.sync_copy(bnd_h, bndv)
        bv = bndv[...]

        def compact(cur):
            runs = []
            for j in range(CAP // NL):
                kj = dbuf[pl.ds(j * NL, NL)]
                vj = ibuf[pl.ds(j * NL, NL)]
                lid = iota + (j * NL)
                kj = jnp.where(lid < cur, kj, inf)
                kj, vj = plsc.sort_key_val(kj, vj)
                runs.append((kj, vj))
            s32 = []
            for j in range(0, CAP // NL, 2):
                ak, av = runs[j]
                bk, bvv = runs[j + 1]
                s32.append(_merge16(ak, av, bk, bvv))
            while len(s32) > 1:
                nxt = []
                for j in range(0, len(s32), 2):
                    nxt.append(_merge32_low(s32[j], s32[j + 1]))
                s32 = nxt
            f0k, f0v, f1k, f1v = s32[0]
            dbuf[pl.ds(0, NL)] = f0k
            dbuf[pl.ds(NL, NL)] = f1k
            ibuf[pl.ds(0, NL)] = f0v
            ibuf[pl.ds(NL, NL)] = f1v
            return f0k, f0v, f1k, f1v

        def compact_state(cur):
            _, _, f1k, _ = compact(cur)
            return jnp.int32(KNN), f1k[NL - 1]

        def mid_body(mi, _):
            xmi = vbx[pl.ds(mi, NL)][0]
            ymi = vby[pl.ds(mi, NL)][0]
            zmi = vbz[pl.ds(mi, NL)][0]
            p2i = vp2[pl.ds(mi, NL)][0]
            bi = mb[pl.ds(mi, NL)][0]
            s = _lane_bcast(bv, bi)[0]
            e = _lane_bcast(bv, bi + 8)[0]
            c0 = (s // 8) * 8
            nch = (e - c0 + (CHUNK - 1)) // CHUNK

            def chunk_body(ci, st):
                cur, tau = st
                c = c0 + ci * CHUNK
                pltpu.sync_copy(bxs_h.at[pl.ds(c, CHUNK)], cx)
                pltpu.sync_copy(bys_h.at[pl.ds(c, CHUNK)], cy)
                pltpu.sync_copy(bzs_h.at[pl.ds(c, CHUNK)], cz)
                pltpu.sync_copy(n2s_h.at[pl.ds(c, CHUNK)], cn)

                def grp(g, st2):
                    cur2, tau2 = st2
                    off = g * NL
                    xv = cx[pl.ds(off, NL)]
                    yv = cy[pl.ds(off, NL)]
                    zv = cz[pl.ds(off, NL)]
                    nv = cn[pl.ds(off, NL)]
                    dot = xv * xmi + yv * ymi + zv * zmi
                    d2 = (p2i + nv) - 2.0 * dot
                    gi = iota + (c + off)
                    qual = (d2 < tau2) & (gi >= s) & (gi < e)
                    p = plsc.all_reduce_population_count(qual)[0]

                    @pl.when(p > 0)
                    def _():
                        plsc.store_compressed(dbuf.at[pl.ds(cur2, NL)], d2,
                                              mask=qual)
                        plsc.store_compressed(ibuf.at[pl.ds(cur2, NL)], gi,
                                              mask=qual)

                    cur3 = cur2 + p
                    return lax.cond(cur3 > CAP - NL,
                                    lambda: compact_state(cur3),
                                    lambda: (cur3, tau2))

                return lax.fori_loop(0, CHUNK // NL, grp, (cur, tau))

            cur, tau = lax.fori_loop(0, nch, chunk_body,
                                     (jnp.int32(0), inf))
            f0k, f0v, f1k, f1v = compact(cur)
            pos_lo = iota * MPW + mi
            pos_hi = (iota + NL) * MPW + mi
            plsc.store_scatter(topd, [pos_lo], f0k)
            plsc.store_scatter(topd, [pos_hi], f1k)
            plsc.store_scatter(topi, [pos_lo], f0v)
            plsc.store_scatter(topi, [pos_hi], f1v)
            return 0

        lax.fori_loop(0, MPW, mid_body, 0)

        for j in range(KNN * MPW // 128):
            idxs = topi.at[pl.ds(j * 128, 128)]
            a = pltpu.async_copy(xs_h.at[idxs], gx.at[pl.ds(j * 128, 128)],
                                 sem1)
            b = pltpu.async_copy(ys_h.at[idxs], gy.at[pl.ds(j * 128, 128)],
                                 sem1)
            c = pltpu.async_copy(zs_h.at[idxs], gz.at[pl.ds(j * 128, 128)],
                                 sem1)
            a.wait()
            b.wait()
            c.wait()
        for k in range(KNN):
            idxs = topi.at[pl.ds(k * MPW, MPW)]
            pltpu.async_copy(xd_h.at[idxs],
                             efx_stage.at[pl.ds(k * MPW, MPW), :],
                             sem2).wait()

        zero = jnp.zeros((NL,), i32)

        def dp_body(g, _):
            off = g * NL
            goff = (g % (MPW // NL)) * NL
            gxv = gx[pl.ds(off, NL)]
            gyv = gy[pl.ds(off, NL)]
            gzv = gz[pl.ds(off, NL)]
            mxv = mx[pl.ds(goff, NL)]
            myv = my[pl.ds(goff, NL)]
            mzv = mz[pl.ds(goff, NL)]
            d2v = topd[pl.ds(off, NL)]
            rows = iota + off
            plsc.store_scatter(efr_stage, [rows, zero], gxv - mxv)
            plsc.store_scatter(efr_stage, [rows, zero + 1], gyv - myv)
            plsc.store_scatter(efr_stage, [rows, zero + 2], gzv - mzv)
            plsc.store_scatter(efr_stage, [rows, zero + 3], d2v)
            return 0

        lax.fori_loop(0, KNN * MPW // NL, dp_body, 0)

        def out_body(k, _):
            pltpu.sync_copy(efx_stage.at[pl.ds(k * MPW, MPW), :],
                            efx_h.at[pl.ds(k * NM + mbase, MPW), :])
            pltpu.sync_copy(efr_stage.at[pl.ds(k * MPW, MPW), :],
                            efr_h.at[pl.ds(k * NM + mbase, MPW), :])
            pltpu.sync_copy(topi.at[pl.ds(k * MPW, MPW)],
                            ti_h.at[pl.ds(k * NM + mbase, MPW)])
            pltpu.sync_copy(topd.at[pl.ds(k * MPW, MPW)],
                            td_h.at[pl.ds(k * NM + mbase, MPW)])
            return 0

        lax.fori_loop(0, KNN, out_body, 0)

    return knn(bxs, bys, bzs, n2s, xs, ys, zs, xm, ym, zm,
               mbx, mby, mbz, mp2, bm, bnd, xd)


def _tc_body(efx_ref, efr_ref, w1x_ref, w1r_ref, b1_ref, g1_ref, be1_ref,
             w2_ref, b2_ref, g2_ref, be2_ref, out_ref,
             h1buf, s1r, s2r, swr, mr, scr, vcr):
    f32 = jnp.float32
    i = pl.program_id(0)
    k = i % KNN
    eps = f32(1e-5)
    neginf = f32(-jnp.inf)
    er = efr_ref[...]
    w = (er[:, 3:4] <= RADSQ).astype(f32)

    @pl.when(i == 0)
    def _init():
        s1r[...] = jnp.zeros((1, OCH), f32)
        s2r[...] = jnp.zeros((1, OCH), f32)
        swr[...] = jnp.zeros((1, 1), f32)

    @pl.when(i < KNN)
    def _pass_a():
        h = jnp.dot(efx_ref[...], w1x_ref[...], preferred_element_type=f32)
        h = h + jnp.dot(er, w1r_ref[...], preferred_element_type=f32)
        h = jnp.maximum(h + b1_ref[...], 0.0)
        h1buf[pl.ds(k * NM, NM), :] = h
        s1r[...] += jnp.sum(h * w, axis=0, keepdims=True)
        s2r[...] += jnp.sum(h * h * w, axis=0, keepdims=True)
        swr[...] += jnp.sum(w, keepdims=True)

    @pl.when(i == KNN)
    def _stats1():
        cnt = jnp.maximum(swr[0, 0], 1.0)
        m = s1r[...] / cnt
        v = jnp.maximum(s2r[...] / cnt - m * m, 0.0)
        mr[...] = m
        scr[...] = lax.rsqrt(v + eps) * g1_ref[...]
        s1r[...] = jnp.zeros((1, OCH), f32)
        s2r[...] = jnp.zeros((1, OCH), f32)

    @pl.when((i >= KNN) & (i < 2 * KNN))
    def _pass_b():
        h = h1buf[pl.ds(k * NM, NM), :]
        h = (h - mr[...]) * scr[...] + be1_ref[...]
        h = jnp.dot(h, w2_ref[...], preferred_element_type=f32)
        h = jnp.maximum(h + b2_ref[...], 0.0)
        h1buf[pl.ds(k * NM, NM), :] = h
        s1r[...] += jnp.sum(h * w, axis=0, keepdims=True)
        s2r[...] += jnp.sum(h * h * w, axis=0, keepdims=True)

    @pl.when(i == 2 * KNN)
    def _stats2():
        cnt = jnp.maximum(swr[0, 0], 1.0)
        m = s1r[...] / cnt
        v = jnp.maximum(s2r[...] / cnt - m * m, 0.0)
        mr[...] = m
        scr[...] = lax.rsqrt(v + eps) * g2_ref[...]

    @pl.when(i >= 2 * KNN)
    def _pass_c():
        h = h1buf[pl.ds(k * NM, NM), :]
        h = (h - mr[...]) * scr[...] + be2_ref[...]
        hm = jnp.where(w > 0, h, neginf)
        base = jnp.where(k == 0, jnp.full((NM, OCH), neginf, f32),
                         out_ref[...])
        out_ref[...] = jnp.maximum(base, hm)
        vcr[...] = jnp.where(k == 0, w, vcr[...] + w)

    @pl.when(i == 3 * KNN - 1)
    def _finish():
        out_ref[...] = jnp.where(vcr[...] > 0, out_ref[...], 0.0)


def kernel(x_dense, pos_dense, pos_mid, batch_dense, batch_mid,
           W1, b1, g1, be1, W2, b2, g2, be2):
    f32 = jnp.float32
    i32 = jnp.int32
    pad = jnp.zeros((CHUNK,), f32)

    def bf(a):
        u = lax.bitcast_convert_type(a, jnp.uint32)
        u = (u + jnp.uint32(0x7FFF) + ((u >> 16) & jnp.uint32(1))) \
            & jnp.uint32(0xFFFF0000)
        return lax.bitcast_convert_type(u, f32)
    xs = jnp.concatenate([pos_dense[:, 0], pad])
    ys = jnp.concatenate([pos_dense[:, 1], pad])
    zs = jnp.concatenate([pos_dense[:, 2], pad])
    n2d = (pos_dense ** 2).sum(1)
    pm2 = (pos_mid ** 2).sum(1)
    bxs = jnp.concatenate([bf(pos_dense[:, 0]), pad])
    bys = jnp.concatenate([bf(pos_dense[:, 1]), pad])
    bzs = jnp.concatenate([bf(pos_dense[:, 2]), pad])
    n2s = jnp.concatenate([n2d, pad])
    xm = pos_mid[:, 0]
    ym = pos_mid[:, 1]
    zm = pos_mid[:, 2]
    mbx = bf(xm)
    mby = bf(ym)
    mbz = bf(zm)
    batches = jnp.arange(NB, dtype=i32)
    bnd = jnp.concatenate([
        jnp.searchsorted(batch_dense, batches, side="left").astype(i32),
        jnp.searchsorted(batch_dense, batches, side="right").astype(i32),
    ])
    efx, efr, _, _ = _sc_knn(bxs, bys, bzs, n2s, xs, ys, zs,
                             xm, ym, zm, mbx, mby, mbz, pm2,
                             batch_mid.astype(i32), bnd, x_dense)

    w1x = W1[:ICH]
    w1r = jnp.concatenate([W1[ICH:], jnp.zeros((1, OCH), f32)])
    kblock = lambda i: (i % KNN, 0)
    whole = lambda i: (0, 0)
    out = pl.pallas_call(
        _tc_body,
        grid=(3 * KNN,),
        in_specs=[
            pl.BlockSpec((NM, ICH), kblock),
            pl.BlockSpec((NM, 4), kblock),
            pl.BlockSpec((ICH, OCH), whole),
            pl.BlockSpec((4, OCH), whole),
            pl.BlockSpec((1, OCH), whole),
            pl.BlockSpec((1, OCH), whole),
            pl.BlockSpec((1, OCH), whole),
            pl.BlockSpec((OCH, OCH), whole),
            pl.BlockSpec((1, OCH), whole),
            pl.BlockSpec((1, OCH), whole),
            pl.BlockSpec((1, OCH), whole),
        ],
        out_specs=pl.BlockSpec((NM, OCH), whole),
        out_shape=jax.ShapeDtypeStruct((NM, OCH), f32),
        scratch_shapes=[
            pltpu.VMEM((KNN * NM, OCH), f32),
            pltpu.VMEM((1, OCH), f32),
            pltpu.VMEM((1, OCH), f32),
            pltpu.VMEM((1, 1), f32),
            pltpu.VMEM((1, OCH), f32),
            pltpu.VMEM((1, OCH), f32),
            pltpu.VMEM((NM, 1), f32),
        ],
    )(efx, efr, w1x, w1r, b1.reshape(1, OCH), g1.reshape(1, OCH),
      be1.reshape(1, OCH), W2, b2.reshape(1, OCH), g2.reshape(1, OCH),
      be2.reshape(1, OCH))
    return out
